# baseline (speedup 1.0000x reference)
import functools

import jax
import jax.numpy as jnp
from jax import lax
from jax.experimental import pallas as pl
from jax.experimental.pallas import tpu as pltpu
from jax.experimental.pallas import tpu_sc as plsc

DIM = 768
B = 4
S = 8192
H = DIM // 2
Q = DIM // 4
NT = S // 2
SBLK = 2048
NBITS = 10
NBINS = 1 << NBITS
DMASK = NBINS - 1
LSTRIDE = S // 16
NW = 32
RPW = B * NT // NW
CH = 64
import numpy as _np

RINV_H = _np.float32(0.00260416674)
RINV_NT = _np.float32(1.0 / 4096.0)
RMAGIC = _np.float32(12582912.0)

_sc_mesh = plsc.VectorSubcoreMesh(core_axis_name="c", subcore_axis_name="s")
_sc_params = pltpu.CompilerParams(needs_layout_passes=False)


def _score_body(x_ref, W1_ref, b1_ref, g_ref, be_ref, w2_ref, b2_ref, o_ref):
    x = x_ref[0]
    dn = (((1,), (1,)), ((), ()))
    hT = lax.dot_general(W1_ref[...], x, dn, preferred_element_type=jnp.float32)
    hT = hT + b1_ref[...][:, None]
    mu = jnp.sum(hT, axis=0, keepdims=True) * RINV_H
    cen = hT - mu
    var = jnp.sum(cen * cen, axis=0, keepdims=True) * RINV_H
    rsd = jnp.sqrt(var + 1e-5)
    hn = cen / rsd * g_ref[...][:, None] + be_ref[...][:, None]
    r = jnp.maximum(hn, 0.0)
    w2 = w2_ref[...]
    cd = (((1,), (0,)), ((), ()))
    z = (lax.dot_general(w2[:, 0:128], r[0:128], cd, preferred_element_type=jnp.float32)
         + lax.dot_general(w2[:, 128:256], r[128:256], cd, preferred_element_type=jnp.float32)
         + lax.dot_general(w2[:, 256:384], r[256:384], cd, preferred_element_type=jnp.float32))
    z = z + b2_ref[...][0]
    s = 1.0 / (1.0 + jnp.exp(-z))
    key = jnp.int32(0x3F800000) - lax.bitcast_convert_type(s, jnp.int32)
    o_ref[...] = key.reshape(1, SBLK // 128, 128)


def _score_keys(x, W1, b1, ln_g, ln_b, W2, b2):
    out = pl.pallas_call(
        _score_body,
        out_shape=jax.ShapeDtypeStruct((B, S // 128, 128), jnp.int32),
        grid=(B, S // SBLK),
        in_specs=[
            pl.BlockSpec((1, SBLK, DIM), lambda b, s: (b, s, 0)),
            pl.BlockSpec((H, DIM), lambda b, s: (0, 0)),
            pl.BlockSpec((H,), lambda b, s: (0,)),
            pl.BlockSpec((H,), lambda b, s: (0,)),
            pl.BlockSpec((H,), lambda b, s: (0,)),
            pl.BlockSpec((1, H), lambda b, s: (0, 0)),
            pl.BlockSpec((1,), lambda b, s: (0,)),
        ],
        out_specs=pl.BlockSpec((1, SBLK // 128, 128), lambda b, s: (b, s, 0)),
    )(x, W1, b1, ln_g, ln_b, W2, b2)
    return out.reshape(B, S)


def _sort_body(keys_hbm, idx_hbm, k0, v0, k1, v1, cnt):
    wid = lax.axis_index("s") * 2 + lax.axis_index("c")

    @pl.when(wid < B)
    def _():
        b = wid
        pltpu.sync_copy(keys_hbm.at[b], k0)
        lane = lax.broadcasted_iota(jnp.int32, (16,), 0)
        gbase = b * S

        def one_pass(shift, src_k, src_v, dst_k, dst_v, first):
            def zero_body(i):
                cnt[pl.ds(i * 16, 16)] = jnp.zeros((16,), jnp.int32)
            pl.loop(0, NBINS)(zero_body)

            def hist_body(i):
                k = plsc.load_gather(src_k, [i + lane * LSTRIDE])
                d = lax.shift_right_logical(k, shift) & DMASK
                ci = d * 16 + lane
                c = plsc.load_gather(cnt, [ci])
                plsc.store_scatter(cnt, [ci], c + 1)
            pl.loop(0, LSTRIDE)(hist_body)

            def pfx_body(i, carry):
                v = cnt[pl.ds(i * 16, 16)]
                c = plsc.cumsum(v)
                cnt[pl.ds(i * 16, 16)] = c - v + carry
                return carry + lax.reduce_sum(v, (0,))
            lax.fori_loop(0, NBINS, pfx_body, jnp.int32(0), unroll=False)

            def scat_body(i):
                src_i = i + lane * LSTRIDE
                k = plsc.load_gather(src_k, [src_i])
                if first:
                    v = src_i + gbase
                else:
                    v = plsc.load_gather(src_v, [src_i])
                d = lax.shift_right_logical(k, shift) & DMASK
                ci = d * 16 + lane
                pos = plsc.load_gather(cnt, [ci])
                plsc.store_scatter(dst_k, [pos], k)
                plsc.store_scatter(dst_v, [pos], v)
                plsc.store_scatter(cnt, [ci], pos + 1)
            pl.loop(0, LSTRIDE)(scat_body)

        one_pass(0, k0, k0, k1, v1, True)
        one_pass(NBITS, k1, v1, k0, v0, False)
        one_pass(2 * NBITS, k0, v0, k1, v1, False)
        pltpu.sync_copy(v1.at[pl.ds(0, NT)], idx_hbm.at[b])


_sort_topk = functools.partial(
    pl.kernel, mesh=_sc_mesh, compiler_params=_sc_params,
    out_type=jax.ShapeDtypeStruct((B, NT), jnp.int32),
    scratch_types=[pltpu.VMEM((S,), jnp.int32),
                   pltpu.VMEM((S,), jnp.int32),
                   pltpu.VMEM((S,), jnp.int32),
                   pltpu.VMEM((S,), jnp.int32),
                   pltpu.VMEM((NBINS * 16,), jnp.int32)],
)(_sort_body)


def _greduce_body(xf_hbm, idx_hbm, part_hbm, idxv, rows, acc, sem):
    wid = lax.axis_index("s") * 2 + lax.axis_index("c")
    pltpu.sync_copy(idx_hbm.at[pl.ds(wid * RPW, RPW)], idxv)

    def zero_body(j):
        acc[0, pl.ds(j * 16, 16)] = jnp.zeros((16,), jnp.float32)
        acc[1, pl.ds(j * 16, 16)] = jnp.zeros((16,), jnp.float32)
    pl.loop(0, DIM // 16)(zero_body)

    def chunk_body(c):
        pltpu.async_copy(xf_hbm.at[idxv.at[pl.ds(c * CH, CH)]], rows, sem).wait()

        def row_body(r):
            for j in range(DIM // 16):
                ds = pl.ds(j * 16, 16)
                v = rows[r, ds]
                acc[0, ds] += v
                acc[1, ds] = jnp.maximum(acc[1, ds], jnp.abs(v))
        pl.loop(0, CH)(row_body)
    pl.loop(0, RPW // CH)(chunk_body)
    pltpu.sync_copy(acc, part_hbm.at[wid])


_gather_reduce = functools.partial(
    pl.kernel, mesh=_sc_mesh, compiler_params=_sc_params,
    out_type=jax.ShapeDtypeStruct((NW, 2, DIM), jnp.float32),
    scratch_types=[pltpu.VMEM((RPW,), jnp.int32),
                   pltpu.VMEM((CH, DIM), jnp.float32),
                   pltpu.VMEM((2, DIM), jnp.float32),
                   pltpu.SemaphoreType.DMA],
)(_greduce_body)


def _final_body(part_ref, Wq1_ref, bq1_ref, Wq2_ref, bq2_ref, step_ref):
    p = part_ref[...]
    p4 = p.reshape(B, NW // B, 2, DIM)
    pooled = jnp.sum(p4[:, :, 0, :], axis=1) * RINV_NT
    scale = jnp.max(p4[:, :, 1, :], axis=1)
    dn = (((1,), (1,)), ((), ()))
    qh = jnp.maximum(
        lax.dot_general(pooled, Wq1_ref[...], dn, preferred_element_type=jnp.float32)
        + bq1_ref[...][None, :], 0.0)
    logits = (lax.dot_general(qh, Wq2_ref[...], dn, preferred_element_type=jnp.float32)
              + bq2_ref[...][None, :])
    maxv = jnp.max(logits, axis=1, keepdims=True)
    io = lax.broadcasted_iota(jnp.int32, (B, 8), 1)
    bits = jnp.min(jnp.where(logits == maxv, io, 8), axis=1) + 1
    pw = jnp.exp2(bits.astype(jnp.float32))
    step_ref[...] = scale / pw[:, None]


def _finalize(partials, Wq1, bq1, Wq2, bq2):
    return pl.pallas_call(
        _final_body,
        out_shape=jax.ShapeDtypeStruct((B, DIM), jnp.float32),
    )(partials, Wq1, bq1, Wq2, bq2)


def _gquant_body(xf_hbm, idx_hbm, step_hbm, out_hbm, idxv, rows, stepv, sem):
    wid = lax.axis_index("s") * 2 + lax.axis_index("c")
    b = wid // (NW // B)
    pltpu.sync_copy(idx_hbm.at[pl.ds(wid * RPW, RPW)], idxv)
    pltpu.sync_copy(step_hbm.at[b], stepv)

    def chunk_body(c):
        pltpu.async_copy(xf_hbm.at[idxv.at[pl.ds(c * CH, CH)]], rows, sem).wait()

        for j in range(DIM // 16):
            ds = pl.ds(j * 16, 16)
            sv = stepv[ds]

            def col_body(r, sv=sv, ds=ds):
                q = rows[r, ds] / sv
                qr = (q + RMAGIC) - RMAGIC
                rows[r, ds] = qr * sv
            pl.loop(0, CH)(col_body)
        pltpu.sync_copy(rows, out_hbm.at[pl.ds(wid * RPW + c * CH, CH)])
    pl.loop(0, RPW // CH)(chunk_body)


_gather_quant = functools.partial(
    pl.kernel, mesh=_sc_mesh, compiler_params=_sc_params,
    out_type=jax.ShapeDtypeStruct((B * NT, DIM), jnp.float32),
    scratch_types=[pltpu.VMEM((RPW,), jnp.int32),
                   pltpu.VMEM((CH, DIM), jnp.float32),
                   pltpu.VMEM((DIM,), jnp.float32),
                   pltpu.SemaphoreType.DMA],
)(_gquant_body)


def kernel(x, W1, b1, ln_g, ln_b, W2, b2, Wq1, bq1, Wq2, bq2):
    keys = _score_keys(x, W1, b1, ln_g, ln_b, W2, b2)
    idx = _sort_topk(keys)
    idxf = idx.reshape(B * NT)
    xf = x.reshape(B * S, DIM)
    partials = _gather_reduce(xf, idxf)
    step = _finalize(partials, Wq1, bq1, Wq2, bq2)
    out = _gather_quant(xf, idxf, step)
    return out.reshape(B, NT, DIM)
